# Initial kernel scaffold; baseline (speedup 1.0000x reference)
import jax
import jax.numpy as jnp
from jax.experimental import pallas as pl


def kernel(x, edge_index, edge_weight, W1, b1, W2, b2):
    raise NotImplementedError("write your pallas kernel here")


# speedup vs baseline: 23.4507x; 23.4507x over previous
import functools

import jax
import jax.numpy as jnp
from jax import lax
from jax.experimental import pallas as pl
from jax.experimental.pallas import tpu as pltpu
from jax.experimental.pallas import tpu_sc as plsc

NC = 2
NS = 16
NW = NC * NS
LANES = 16
CH = 1024
NJ = CH // 128


def _flat_worker_id():
  return lax.axis_index("s") * NC + lax.axis_index("c")


def _zero_fill(ref, n_rows, n_vregs):
  z = jnp.zeros((LANES,), jnp.float32)

  def body(i, _):
    if n_vregs == 0:
      ref[pl.ds(i * LANES, LANES)] = z
    else:
      for k in range(n_vregs):
        ref[i, pl.ds(k * LANES, LANES)] = z
    return 0

  count = n_rows // LANES if n_vregs == 0 else n_rows
  lax.fori_loop(0, count, body, 0, unroll=8)


def _deg_body(n_pad, col2_hbm, w_hbm, out_hbm, colb, wb, zb, deg_sh):
  cid = lax.axis_index("c")
  sid = lax.axis_index("s")
  wid = _flat_worker_id()
  rows_per_sub = n_pad // NS
  n_chunks = col2_hbm.shape[0] * 128 // (NW * CH)

  _zero_fill(zb, rows_per_sub, 0)
  pltpu.sync_copy(zb, deg_sh.at[pl.ds(sid * rows_per_sub, rows_per_sub)])
  plsc.subcore_barrier()

  def chunk(t, _):
    base = wid * (n_chunks * CH) + t * CH
    crow = pl.multiple_of(base // 128, 8)
    pltpu.sync_copy(col2_hbm.at[pl.ds(crow, NJ), :], colb)
    pltpu.sync_copy(w_hbm.at[pl.ds(base, CH)], wb)
    for j in range(NJ):
      pltpu.sync_copy(wb.at[pl.ds(j * 128, 128)], deg_sh.at[colb.at[j]],
                      add=True)
    return 0

  lax.fori_loop(0, n_chunks, chunk, 0)
  plsc.subcore_barrier()
  sl = pl.ds(sid * rows_per_sub, rows_per_sub)
  pltpu.sync_copy(deg_sh.at[sl], out_hbm.at[cid, sl])


def _agg_body(n_pad, nv, row_hbm, col2_hbm, w_hbm, y_hbm, out_hbm,
              rowb, colb, wb, msgb, acc_sh, sem):
  cid = lax.axis_index("c")
  sid = lax.axis_index("s")
  wid = _flat_worker_id()
  rows_per_sub = n_pad // NS
  n_chunks = row_hbm.shape[0] // (NW * CH)

  _zero_fill(msgb, CH, nv)
  sl = pl.ds(sid * rows_per_sub, rows_per_sub)
  pltpu.sync_copy(msgb.at[pl.ds(0, rows_per_sub), :], acc_sh.at[sl, :])
  plsc.subcore_barrier()

  def chunk(t, _):
    base = wid * (n_chunks * CH) + t * CH
    crow = pl.multiple_of(base // 128, 8)
    pltpu.sync_copy(row_hbm.at[pl.ds(base, CH)], rowb)
    pltpu.sync_copy(col2_hbm.at[pl.ds(crow, NJ), :], colb)
    pltpu.sync_copy(w_hbm.at[pl.ds(base, CH)], wb)
    pltpu.async_copy(y_hbm.at[rowb], msgb, sem).wait()

    def scale(g, _):
      wv = wb[pl.ds(g * LANES, LANES)]
      for u in range(LANES):
        i = g * LANES + u
        for k in range(nv):
          s = pl.ds(k * LANES, LANES)
          msgb[i, s] = msgb[i, s] * wv[u]
      return 0

    lax.fori_loop(0, CH // LANES, scale, 0)
    for j in range(NJ):
      pltpu.sync_copy(msgb.at[pl.ds(j * 128, 128), :], acc_sh.at[colb.at[j]],
                      add=True)
    return 0

  lax.fori_loop(0, n_chunks, chunk, 0)
  plsc.subcore_barrier()
  pltpu.sync_copy(acc_sh.at[sl, :], out_hbm.at[cid, sl, :])


def _sc_deg(col2, w, n_pad):
  mesh = plsc.VectorSubcoreMesh(core_axis_name="c", subcore_axis_name="s")
  return pl.kernel(
      functools.partial(_deg_body, n_pad),
      out_type=jax.ShapeDtypeStruct((NC, n_pad), jnp.float32),
      mesh=mesh,
      scratch_types=[
          pltpu.VMEM((NJ, 128), jnp.int32),
          pltpu.VMEM((CH,), jnp.float32),
          pltpu.VMEM((n_pad // NS,), jnp.float32),
          pltpu.VMEM_SHARED((n_pad,), jnp.float32),
      ],
  )(col2, w)


def _sc_agg(row, col2, w, y, n_pad, feat):
  nv = feat // LANES
  mesh = plsc.VectorSubcoreMesh(core_axis_name="c", subcore_axis_name="s")
  return pl.kernel(
      functools.partial(_agg_body, n_pad, nv),
      out_type=jax.ShapeDtypeStruct((NC, n_pad, feat), jnp.float32),
      mesh=mesh,
      scratch_types=[
          pltpu.VMEM((CH,), jnp.int32),
          pltpu.VMEM((NJ, 128), jnp.int32),
          pltpu.VMEM((CH,), jnp.float32),
          pltpu.VMEM((CH, feat), jnp.float32),
          pltpu.VMEM_SHARED((n_pad, feat), jnp.float32),
          pltpu.SemaphoreType.DMA,
      ],
      compiler_params=pltpu.CompilerParams(use_tc_tiling_on_sc=False),
  )(row, col2, w, y)


def _tc_a_body(deg_ref, x_ref, w1_ref, dinv_ref, y1_ref):
  deg = deg_ref[0] + deg_ref[1] + 1.0
  dinv = lax.rsqrt(deg)
  dinv_ref[...] = dinv
  xw = jnp.dot(x_ref[...], w1_ref[...], preferred_element_type=jnp.float32)
  y1_ref[...] = xw * dinv


def _tc_b_body(p1_ref, y1_ref, dinv_ref, b1_ref, w2_ref, y2_ref):
  dinv = dinv_ref[...]
  agg = p1_ref[0] + p1_ref[1] + y1_ref[...]
  h = jnp.maximum(agg * dinv + b1_ref[...], 0.0)
  hw = jnp.dot(h, w2_ref[...], preferred_element_type=jnp.float32)
  y2_ref[...] = hw * dinv


def _tc_c_body(p2_ref, y2_ref, dinv_ref, b2_ref, out_ref):
  agg = p2_ref[0] + p2_ref[1] + y2_ref[...]
  out_ref[...] = agg * dinv_ref[...] + b2_ref[...]


def kernel(x, edge_index, edge_weight, W1, b1, W2, b2):
  n, _ = x.shape
  e = edge_index.shape[1]
  hid = W1.shape[1]
  ncls = W2.shape[1]

  n_pad = ((n + NW * LANES - 1) // (NW * LANES)) * (NW * LANES)
  e_pad = ((e + NW * CH - 1) // (NW * CH)) * (NW * CH)
  fpad2 = ((ncls + LANES - 1) // LANES) * LANES

  row = jnp.concatenate([edge_index[0],
                         jnp.zeros((e_pad - e,), jnp.int32)])
  colp = jnp.concatenate([edge_index[1],
                          jnp.zeros((e_pad - e,), jnp.int32)])
  col2 = colp.reshape(e_pad // 128, 128)
  w = jnp.concatenate([edge_weight, jnp.zeros((e_pad - e,), jnp.float32)])
  xp = jnp.concatenate([x, jnp.zeros((n_pad - n, x.shape[1]), jnp.float32)])
  w2p = jnp.concatenate(
      [W2, jnp.zeros((hid, fpad2 - ncls), jnp.float32)], axis=1)
  b1r = b1.reshape(1, hid)
  b2r = jnp.concatenate([b2, jnp.zeros((fpad2 - ncls,), jnp.float32)])
  b2r = b2r.reshape(1, fpad2)

  deg = _sc_deg(col2, w, n_pad)
  deg3 = deg.reshape(NC, n_pad, 1)

  dinv, y1 = pl.pallas_call(
      _tc_a_body,
      out_shape=(jax.ShapeDtypeStruct((n_pad, 1), jnp.float32),
                 jax.ShapeDtypeStruct((n_pad, hid), jnp.float32)),
  )(deg3, xp, W1)

  p1 = _sc_agg(row, col2, w, y1, n_pad, hid)

  y2 = pl.pallas_call(
      _tc_b_body,
      out_shape=jax.ShapeDtypeStruct((n_pad, fpad2), jnp.float32),
  )(p1, y1, dinv, b1r, w2p)

  p2 = _sc_agg(row, col2, w, y2, n_pad, fpad2)

  out = pl.pallas_call(
      _tc_c_body,
      out_shape=jax.ShapeDtypeStruct((n_pad, fpad2), jnp.float32),
  )(p2, y2, dinv, b2r)

  return out[:n, :ncls]


# speedup vs baseline: 39.6288x; 1.6899x over previous
import functools

import jax
import jax.numpy as jnp
from jax import lax
from jax.experimental import pallas as pl
from jax.experimental.pallas import tpu as pltpu
from jax.experimental.pallas import tpu_sc as plsc

NC = 2
NS = 16
NW = NC * NS
LANES = 16
CH = 1024
NJ = CH // 128


def _flat_worker_id():
  return lax.axis_index("s") * NC + lax.axis_index("c")


def _zero_fill(ref, n_rows, n_vregs):
  z = jnp.zeros((LANES,), jnp.float32)

  def body(i, _):
    if n_vregs == 0:
      ref[pl.ds(i * LANES, LANES)] = z
    else:
      for k in range(n_vregs):
        ref[i, pl.ds(k * LANES, LANES)] = z
    return 0

  count = n_rows // LANES if n_vregs == 0 else n_rows
  lax.fori_loop(0, count, body, 0, unroll=8)


def _deg_body(n_pad, col2_hbm, w_hbm, out_hbm, colb, wb, zb, deg_sh):
  cid = lax.axis_index("c")
  sid = lax.axis_index("s")
  wid = _flat_worker_id()
  rows_per_sub = n_pad // NS
  n_chunks = col2_hbm.shape[0] * 128 // (NW * CH)

  _zero_fill(zb, rows_per_sub, 0)
  pltpu.sync_copy(zb, deg_sh.at[pl.ds(sid * rows_per_sub, rows_per_sub)])
  plsc.subcore_barrier()

  def chunk(t, _):
    base = wid * (n_chunks * CH) + t * CH
    crow = pl.multiple_of(base // 128, 8)
    pltpu.sync_copy(col2_hbm.at[pl.ds(crow, NJ), :], colb)
    pltpu.sync_copy(w_hbm.at[pl.ds(base, CH)], wb)
    for j in range(NJ):
      pltpu.sync_copy(wb.at[pl.ds(j * 128, 128)], deg_sh.at[colb.at[j]],
                      add=True)
    return 0

  lax.fori_loop(0, n_chunks, chunk, 0)
  plsc.subcore_barrier()
  sl = pl.ds(sid * rows_per_sub, rows_per_sub)
  pltpu.sync_copy(deg_sh.at[sl], out_hbm.at[cid, sl])


def _agg_body(n_pad, n_chunks, row_hbm, col2_hbm, w_hbm, y_hbm, out_hbm,
              row_all, col_all, w_all, msg0, msg1, acc_sh,
              sem_g0, sem_g1, sem_s0, sem_s1):
  cid = lax.axis_index("c")
  sid = lax.axis_index("s")
  wid = _flat_worker_id()
  rows_per_sub = n_pad // NS
  epw = n_chunks * CH
  msgb = (msg0, msg1)
  sem_g = (sem_g0, sem_g1)
  sem_s = (sem_s0, sem_s1)

  base = wid * epw
  crow = pl.multiple_of(base // 128, 8)
  d_row = pltpu.async_copy(row_hbm.at[pl.ds(base, epw)], row_all, sem_g0)
  d_col = pltpu.async_copy(col2_hbm.at[pl.ds(crow, epw // 128), :], col_all,
                           sem_g1)
  d_w = pltpu.async_copy(w_hbm.at[pl.ds(base, epw)], w_all, sem_s0)

  _zero_fill(msg0, rows_per_sub, 1)
  sl = pl.ds(sid * rows_per_sub, rows_per_sub)
  pltpu.sync_copy(msg0.at[pl.ds(0, rows_per_sub), :], acc_sh.at[sl, :])
  d_row.wait()
  d_col.wait()
  d_w.wait()
  plsc.subcore_barrier()

  def issue_gather(t):
    slot = t % 2
    return pltpu.async_copy(y_hbm.at[row_all.at[pl.ds(t * CH, CH)]],
                            msgb[slot], sem_g[slot])

  def scale(t):
    slot = t % 2
    buf = msgb[slot]

    def body(g, _):
      wv = w_all[pl.ds(t * CH + g * LANES, LANES)]
      for u in range(LANES):
        i = g * LANES + u
        buf[i, :] = buf[i, :] * wv[u]
      return 0

    lax.fori_loop(0, CH // LANES, body, 0)

  def issue_scatter(t):
    slot = t % 2
    ds = []
    for jj in range(NJ):
      ds.append(pltpu.async_copy(
          msgb[slot].at[pl.ds(jj * 128, 128), :],
          acc_sh.at[col_all.at[t * NJ + jj]], sem_s[slot], add=True))
    return ds

  gath = {0: issue_gather(0)}
  scat = {}
  for t in range(n_chunks):
    if t + 1 < n_chunks:
      if t - 1 >= 0:
        for d in scat.pop(t - 1):
          d.wait()
      gath[t + 1] = issue_gather(t + 1)
    gath.pop(t).wait()
    scale(t)
    scat[t] = issue_scatter(t)
  for ds in scat.values():
    for d in ds:
      d.wait()

  plsc.subcore_barrier()
  pltpu.sync_copy(acc_sh.at[sl, :], out_hbm.at[cid, sl, :])


def _sc_deg(col2, w, n_pad):
  mesh = plsc.VectorSubcoreMesh(core_axis_name="c", subcore_axis_name="s")
  return pl.kernel(
      functools.partial(_deg_body, n_pad),
      out_type=jax.ShapeDtypeStruct((NC, n_pad), jnp.float32),
      mesh=mesh,
      scratch_types=[
          pltpu.VMEM((NJ, 128), jnp.int32),
          pltpu.VMEM((CH,), jnp.float32),
          pltpu.VMEM((n_pad // NS,), jnp.float32),
          pltpu.VMEM_SHARED((n_pad,), jnp.float32),
      ],
  )(col2, w)


def _sc_agg(row, col2, w, y, n_pad, feat):
  e_pad = row.shape[0]
  n_chunks = e_pad // (NW * CH)
  epw = n_chunks * CH
  mesh = plsc.VectorSubcoreMesh(core_axis_name="c", subcore_axis_name="s")
  return pl.kernel(
      functools.partial(_agg_body, n_pad, n_chunks),
      out_type=jax.ShapeDtypeStruct((NC, n_pad, feat), jnp.float32),
      mesh=mesh,
      scratch_types=[
          pltpu.VMEM((epw,), jnp.int32),
          pltpu.VMEM((epw // 128, 128), jnp.int32),
          pltpu.VMEM((epw,), jnp.float32),
          pltpu.VMEM((CH, feat), jnp.float32),
          pltpu.VMEM((CH, feat), jnp.float32),
          pltpu.VMEM_SHARED((n_pad, feat), jnp.float32),
          pltpu.SemaphoreType.DMA,
          pltpu.SemaphoreType.DMA,
          pltpu.SemaphoreType.DMA,
          pltpu.SemaphoreType.DMA,
      ],
      compiler_params=pltpu.CompilerParams(use_tc_tiling_on_sc=False),
  )(row, col2, w, y)


def _tc_a_body(deg_ref, x_ref, w1_ref, dinv_ref, y1_ref):
  deg = deg_ref[0] + deg_ref[1] + 1.0
  dinv = lax.rsqrt(deg)
  dinv_ref[...] = dinv
  xw = jnp.dot(x_ref[...], w1_ref[...], preferred_element_type=jnp.float32)
  y1_ref[...] = xw * dinv


def _tc_b_body(p1_ref, y1_ref, dinv_ref, b1_ref, z_ref):
  dinv = dinv_ref[...]
  agg = p1_ref[0] + p1_ref[1] + y1_ref[...]
  h = jnp.maximum(agg * dinv + b1_ref[...], 0.0)
  z_ref[...] = h * dinv


def _tc_c_body(p2_ref, z_ref, dinv_ref, b2_ref, w2_ref, out_ref):
  agg = p2_ref[0] + p2_ref[1] + z_ref[...]
  aw = jnp.dot(agg, w2_ref[...], preferred_element_type=jnp.float32)
  out_ref[...] = aw * dinv_ref[...] + b2_ref[...]


def kernel(x, edge_index, edge_weight, W1, b1, W2, b2):
  n, _ = x.shape
  e = edge_index.shape[1]
  hid = W1.shape[1]
  ncls = W2.shape[1]

  n_pad = ((n + NW * LANES - 1) // (NW * LANES)) * (NW * LANES)
  e_pad = ((e + NW * CH - 1) // (NW * CH)) * (NW * CH)

  row = jnp.concatenate([edge_index[0],
                         jnp.zeros((e_pad - e,), jnp.int32)])
  colp = jnp.concatenate([edge_index[1],
                          jnp.zeros((e_pad - e,), jnp.int32)])
  col2 = colp.reshape(e_pad // 128, 128)
  w = jnp.concatenate([edge_weight, jnp.zeros((e_pad - e,), jnp.float32)])
  xp = jnp.concatenate([x, jnp.zeros((n_pad - n, x.shape[1]), jnp.float32)])
  b1r = b1.reshape(1, hid)
  b2r = b2.reshape(1, ncls)

  deg = _sc_deg(col2, w, n_pad)
  deg3 = deg.reshape(NC, n_pad, 1)

  dinv, y1 = pl.pallas_call(
      _tc_a_body,
      out_shape=(jax.ShapeDtypeStruct((n_pad, 1), jnp.float32),
                 jax.ShapeDtypeStruct((n_pad, hid), jnp.float32)),
  )(deg3, xp, W1)

  p1 = _sc_agg(row, col2, w, y1, n_pad, hid)

  z = pl.pallas_call(
      _tc_b_body,
      out_shape=jax.ShapeDtypeStruct((n_pad, hid), jnp.float32),
  )(p1, y1, dinv, b1r)

  p2 = _sc_agg(row, col2, w, z, n_pad, hid)

  out = pl.pallas_call(
      _tc_c_body,
      out_shape=jax.ShapeDtypeStruct((n_pad, ncls), jnp.float32),
  )(p2, z, dinv, b2r, W2)

  return out[:n, :ncls]


# speedup vs baseline: 55.2839x; 1.3950x over previous
import functools

import jax
import jax.numpy as jnp
from jax import lax
from jax.experimental import pallas as pl
from jax.experimental.pallas import tpu as pltpu
from jax.experimental.pallas import tpu_sc as plsc

NC = 2
NS = 16
NW = NC * NS
LANES = 16
CH = 1024
NJ = CH // 128


def _flat_worker_id():
  return lax.axis_index("s") * NC + lax.axis_index("c")


def _zero_fill(ref, n_rows, n_vregs):
  z = jnp.zeros((LANES,), jnp.float32)

  def body(i, _):
    if n_vregs == 0:
      ref[pl.ds(i * LANES, LANES)] = z
    else:
      for k in range(n_vregs):
        ref[i, pl.ds(k * LANES, LANES)] = z
    return 0

  count = n_rows // LANES if n_vregs == 0 else n_rows
  lax.fori_loop(0, count, body, 0, unroll=8)


def _deg_body(n_pad, n_jrows, col2_hbm, w_hbm, out_hbm, col_all, w_all, zb,
              deg_sh, sem_a, sem_b):
  cid = lax.axis_index("c")
  sid = lax.axis_index("s")
  wid = _flat_worker_id()
  rows_per_sub = n_pad // NS
  epw = n_jrows * 128

  base = wid * epw
  crow = pl.multiple_of(base // 128, 8)
  d_col = pltpu.async_copy(col2_hbm.at[pl.ds(crow, n_jrows), :], col_all,
                           sem_a)
  d_w = pltpu.async_copy(w_hbm.at[pl.ds(base, epw)], w_all, sem_b)

  _zero_fill(zb, rows_per_sub, 0)
  pltpu.sync_copy(zb, deg_sh.at[pl.ds(sid * rows_per_sub, rows_per_sub)])
  d_col.wait()
  d_w.wait()
  plsc.subcore_barrier()

  pend = []
  for j in range(n_jrows):
    if len(pend) >= 16:
      pend.pop(0).wait()
    pend.append(pltpu.async_copy(w_all.at[pl.ds(j * 128, 128)],
                                 deg_sh.at[col_all.at[j]], sem_a, add=True))
  for d in pend:
    d.wait()

  plsc.subcore_barrier()
  sl = pl.ds(sid * rows_per_sub, rows_per_sub)
  pltpu.sync_copy(deg_sh.at[sl], out_hbm.at[cid, sl])


def _agg_body(n_pad, n_chunks, row_hbm, col2_hbm, w_hbm, y_hbm, out_hbm,
              row_all, col_all, w_all, msg0, msg1, y_sh, acc_sh,
              sem_g0, sem_g1, sem_s0, sem_s1):
  cid = lax.axis_index("c")
  sid = lax.axis_index("s")
  wid = _flat_worker_id()
  rows_per_sub = n_pad // NS
  epw = n_chunks * CH
  msgb = (msg0, msg1)
  sem_g = (sem_g0, sem_g1)
  sem_s = (sem_s0, sem_s1)

  base = wid * epw
  crow = pl.multiple_of(base // 128, 8)
  sl = pl.ds(sid * rows_per_sub, rows_per_sub)
  d_row = pltpu.async_copy(row_hbm.at[pl.ds(base, epw)], row_all, sem_g0)
  d_col = pltpu.async_copy(col2_hbm.at[pl.ds(crow, epw // 128), :], col_all,
                           sem_g1)
  d_w = pltpu.async_copy(w_hbm.at[pl.ds(base, epw)], w_all, sem_s0)
  d_y = pltpu.async_copy(y_hbm.at[sl, :], y_sh.at[sl, :], sem_s1)

  _zero_fill(msg0, rows_per_sub, 1)
  pltpu.sync_copy(msg0.at[pl.ds(0, rows_per_sub), :], acc_sh.at[sl, :])
  d_row.wait()
  d_col.wait()
  d_w.wait()
  d_y.wait()
  plsc.subcore_barrier()

  def issue_gather(t):
    slot = t % 2
    return pltpu.async_copy(y_sh.at[row_all.at[pl.ds(t * CH, CH)]],
                            msgb[slot], sem_g[slot])

  def scale(t):
    slot = t % 2
    buf = msgb[slot]

    def body(g, _):
      wv = w_all[pl.ds(t * CH + g * LANES, LANES)]
      for u in range(LANES):
        i = g * LANES + u
        buf[i, :] = buf[i, :] * wv[u]
      return 0

    lax.fori_loop(0, CH // LANES, body, 0)

  def issue_scatter(t):
    slot = t % 2
    ds = []
    for jj in range(NJ):
      ds.append(pltpu.async_copy(
          msgb[slot].at[pl.ds(jj * 128, 128), :],
          acc_sh.at[col_all.at[t * NJ + jj]], sem_s[slot], add=True))
    return ds

  gath = {0: issue_gather(0)}
  scat = {}
  for t in range(n_chunks):
    if t + 1 < n_chunks:
      if t - 1 >= 0:
        for d in scat.pop(t - 1):
          d.wait()
      gath[t + 1] = issue_gather(t + 1)
    gath.pop(t).wait()
    scale(t)
    scat[t] = issue_scatter(t)
  for ds in scat.values():
    for d in ds:
      d.wait()

  plsc.subcore_barrier()
  pltpu.sync_copy(acc_sh.at[sl, :], out_hbm.at[cid, sl, :])


def _sc_deg(col2, w, n_pad):
  e_pad = col2.shape[0] * 128
  mesh = plsc.VectorSubcoreMesh(core_axis_name="c", subcore_axis_name="s")
  return pl.kernel(
      functools.partial(_deg_body, n_pad, e_pad // NW // 128),
      out_type=jax.ShapeDtypeStruct((NC, n_pad), jnp.float32),
      mesh=mesh,
      scratch_types=[
          pltpu.VMEM((e_pad // NW // 128, 128), jnp.int32),
          pltpu.VMEM((e_pad // NW,), jnp.float32),
          pltpu.VMEM((n_pad // NS,), jnp.float32),
          pltpu.VMEM_SHARED((n_pad,), jnp.float32),
          pltpu.SemaphoreType.DMA,
          pltpu.SemaphoreType.DMA,
      ],
  )(col2, w)


def _sc_agg(row, col2, w, y, n_pad, feat):
  e_pad = row.shape[0]
  n_chunks = e_pad // (NW * CH)
  epw = n_chunks * CH
  mesh = plsc.VectorSubcoreMesh(core_axis_name="c", subcore_axis_name="s")
  return pl.kernel(
      functools.partial(_agg_body, n_pad, n_chunks),
      out_type=jax.ShapeDtypeStruct((NC, n_pad, feat), jnp.float32),
      mesh=mesh,
      scratch_types=[
          pltpu.VMEM((epw,), jnp.int32),
          pltpu.VMEM((epw // 128, 128), jnp.int32),
          pltpu.VMEM((epw,), jnp.float32),
          pltpu.VMEM((CH, feat), jnp.float32),
          pltpu.VMEM((CH, feat), jnp.float32),
          pltpu.VMEM_SHARED((n_pad, feat), jnp.float32),
          pltpu.VMEM_SHARED((n_pad, feat), jnp.float32),
          pltpu.SemaphoreType.DMA,
          pltpu.SemaphoreType.DMA,
          pltpu.SemaphoreType.DMA,
          pltpu.SemaphoreType.DMA,
      ],
      compiler_params=pltpu.CompilerParams(use_tc_tiling_on_sc=False),
  )(row, col2, w, y)


def _tc_a_body(deg_ref, x_ref, w1_ref, dinv_ref, y1_ref):
  deg = deg_ref[0] + deg_ref[1] + 1.0
  dinv = lax.rsqrt(deg)
  dinv_ref[...] = dinv
  xw = jnp.dot(x_ref[...], w1_ref[...], preferred_element_type=jnp.float32)
  y1_ref[...] = xw * dinv


def _tc_b_body(p1_ref, y1_ref, dinv_ref, b1_ref, z_ref):
  dinv = dinv_ref[...]
  agg = p1_ref[0] + p1_ref[1] + y1_ref[...]
  h = jnp.maximum(agg * dinv + b1_ref[...], 0.0)
  z_ref[...] = h * dinv


def _tc_c_body(p2_ref, z_ref, dinv_ref, b2_ref, w2_ref, out_ref):
  agg = p2_ref[0] + p2_ref[1] + z_ref[...]
  aw = jnp.dot(agg, w2_ref[...], preferred_element_type=jnp.float32)
  out_ref[...] = aw * dinv_ref[...] + b2_ref[...]


def kernel(x, edge_index, edge_weight, W1, b1, W2, b2):
  n, _ = x.shape
  e = edge_index.shape[1]
  hid = W1.shape[1]
  ncls = W2.shape[1]

  n_pad = ((n + NW * LANES - 1) // (NW * LANES)) * (NW * LANES)
  e_pad = ((e + NW * CH - 1) // (NW * CH)) * (NW * CH)

  row = jnp.concatenate([edge_index[0],
                         jnp.zeros((e_pad - e,), jnp.int32)])
  colp = jnp.concatenate([edge_index[1],
                          jnp.zeros((e_pad - e,), jnp.int32)])
  col2 = colp.reshape(e_pad // 128, 128)
  w = jnp.concatenate([edge_weight, jnp.zeros((e_pad - e,), jnp.float32)])
  xp = jnp.concatenate([x, jnp.zeros((n_pad - n, x.shape[1]), jnp.float32)])
  b1r = b1.reshape(1, hid)
  b2r = b2.reshape(1, ncls)

  deg = _sc_deg(col2, w, n_pad)
  deg3 = deg.reshape(NC, n_pad, 1)

  dinv, y1 = pl.pallas_call(
      _tc_a_body,
      out_shape=(jax.ShapeDtypeStruct((n_pad, 1), jnp.float32),
                 jax.ShapeDtypeStruct((n_pad, hid), jnp.float32)),
  )(deg3, xp, W1)

  p1 = _sc_agg(row, col2, w, y1, n_pad, hid)

  z = pl.pallas_call(
      _tc_b_body,
      out_shape=jax.ShapeDtypeStruct((n_pad, hid), jnp.float32),
  )(p1, y1, dinv, b1r)

  p2 = _sc_agg(row, col2, w, z, n_pad, hid)

  out = pl.pallas_call(
      _tc_c_body,
      out_shape=jax.ShapeDtypeStruct((n_pad, ncls), jnp.float32),
  )(p2, z, dinv, b2r, W2)

  return out[:n, :ncls]


# speedup vs baseline: 60.1143x; 1.0874x over previous
import functools

import jax
import jax.numpy as jnp
from jax import lax
from jax.experimental import pallas as pl
from jax.experimental.pallas import tpu as pltpu
from jax.experimental.pallas import tpu_sc as plsc

NC = 2
NS = 16
NW = NC * NS
LANES = 16
CH = 1024
NJ = CH // 128


def _flat_worker_id():
  return lax.axis_index("s") * NC + lax.axis_index("c")


def _zero_fill(ref, n_rows, n_vregs):
  z = jnp.zeros((LANES,), jnp.float32)

  def body(i, _):
    if n_vregs == 0:
      ref[pl.ds(i * LANES, LANES)] = z
    else:
      for k in range(n_vregs):
        ref[i, pl.ds(k * LANES, LANES)] = z
    return 0

  count = n_rows // LANES if n_vregs == 0 else n_rows
  lax.fori_loop(0, count, body, 0, unroll=8)


def _deg_body(n_pad, n_jrows, col2_hbm, w_hbm, out_hbm, col_all, w_all, zb,
              deg_sh, sem_a, sem_b):
  cid = lax.axis_index("c")
  sid = lax.axis_index("s")
  wid = _flat_worker_id()
  rows_per_sub = n_pad // NS
  epw = n_jrows * 128

  base = wid * epw
  crow = pl.multiple_of(base // 128, 8)
  d_col = pltpu.async_copy(col2_hbm.at[pl.ds(crow, n_jrows), :], col_all,
                           sem_a)
  d_w = pltpu.async_copy(w_hbm.at[pl.ds(base, epw)], w_all, sem_b)

  _zero_fill(zb, rows_per_sub, 0)
  pltpu.sync_copy(zb, deg_sh.at[pl.ds(sid * rows_per_sub, rows_per_sub)])
  d_col.wait()
  d_w.wait()
  plsc.subcore_barrier()

  pend = []
  for j in range(n_jrows):
    if len(pend) >= 16:
      pend.pop(0).wait()
    pend.append(pltpu.async_copy(w_all.at[pl.ds(j * 128, 128)],
                                 deg_sh.at[col_all.at[j]], sem_a, add=True))
  for d in pend:
    d.wait()

  plsc.subcore_barrier()
  sl = pl.ds(sid * rows_per_sub, rows_per_sub)
  pltpu.sync_copy(deg_sh.at[sl], out_hbm.at[cid, sl])


def _agg_body(n_pad, n_chunks, row_hbm, col2_hbm, w_hbm, y_hbm, out_hbm,
              row_all, col_all, w_all, msg0, msg1, y_sh, acc_sh,
              sem_g0, sem_g1, sem_s0, sem_s1):
  cid = lax.axis_index("c")
  sid = lax.axis_index("s")
  wid = _flat_worker_id()
  rows_per_sub = n_pad // NS
  epw = n_chunks * CH
  msgb = (msg0, msg1)
  sem_g = (sem_g0, sem_g1)
  sem_s = (sem_s0, sem_s1)

  base = wid * epw
  crow = pl.multiple_of(base // 128, 8)
  sl = pl.ds(sid * rows_per_sub, rows_per_sub)
  d_row = pltpu.async_copy(row_hbm.at[pl.ds(base, epw)], row_all, sem_g0)
  d_col = pltpu.async_copy(col2_hbm.at[pl.ds(crow, epw // 128), :], col_all,
                           sem_g1)
  d_w = pltpu.async_copy(w_hbm.at[pl.ds(base, epw)], w_all, sem_s0)
  d_y = pltpu.async_copy(y_hbm.at[sl, :], y_sh.at[sl, :], sem_s1)

  _zero_fill(msg0, rows_per_sub, 1)
  pltpu.sync_copy(msg0.at[pl.ds(0, rows_per_sub), :], acc_sh.at[sl, :])
  d_row.wait()
  d_col.wait()
  d_w.wait()
  d_y.wait()
  plsc.subcore_barrier()

  def issue_gather(t):
    slot = t % 2
    return pltpu.async_copy(y_sh.at[row_all.at[pl.ds(t * CH, CH)]],
                            msgb[slot], sem_g[slot])

  def scale(t):
    slot = t % 2
    buf = msgb[slot]

    def body(g, _):
      wv = w_all[pl.ds(t * CH + g * LANES, LANES)]
      for u in range(LANES):
        i = g * LANES + u
        buf[i, :] = buf[i, :] * wv[u]
      return 0

    lax.fori_loop(0, CH // LANES, body, 0)

  def issue_scatter(t):
    slot = t % 2
    ds = []
    for jj in range(NJ):
      ds.append(pltpu.async_copy(
          msgb[slot].at[pl.ds(jj * 128, 128), :],
          acc_sh.at[col_all.at[t * NJ + jj]], sem_s[slot], add=True))
    return ds

  gath = {0: issue_gather(0)}
  scat = {}
  for t in range(n_chunks):
    if t + 1 < n_chunks:
      if t - 1 >= 0:
        for d in scat.pop(t - 1):
          d.wait()
      gath[t + 1] = issue_gather(t + 1)
    gath.pop(t).wait()
    scale(t)
    scat[t] = issue_scatter(t)
  for ds in scat.values():
    for d in ds:
      d.wait()

  plsc.subcore_barrier()
  pltpu.sync_copy(acc_sh.at[sl, :], out_hbm.at[cid, sl, :])


def _agg2_body(n_pad, n_chunks, row_hbm, col2_hbm, w_hbm, p1_hbm, y1_hbm,
               dinv_hbm, b1_hbm, p2_out, z_out,
               row_all, col_all, w_all, msg0, msg1, ab, bb, cb, dvb, b1b,
               y_sh, acc_sh, sem_g0, sem_g1, sem_s0, sem_s1):
  cid = lax.axis_index("c")
  sid = lax.axis_index("s")
  wid = _flat_worker_id()
  rows_per_sub = n_pad // NS
  epw = n_chunks * CH
  msgb = (msg0, msg1)
  sem_g = (sem_g0, sem_g1)
  sem_s = (sem_s0, sem_s1)

  base = wid * epw
  crow = pl.multiple_of(base // 128, 8)
  sl = pl.ds(sid * rows_per_sub, rows_per_sub)
  stag = [
      pltpu.async_copy(row_hbm.at[pl.ds(base, epw)], row_all, sem_g0),
      pltpu.async_copy(col2_hbm.at[pl.ds(crow, epw // 128), :], col_all,
                       sem_g1),
      pltpu.async_copy(w_hbm.at[pl.ds(base, epw)], w_all, sem_s0),
      pltpu.async_copy(p1_hbm.at[0, sl, :], ab, sem_s1),
      pltpu.async_copy(p1_hbm.at[1, sl, :], bb, sem_g0),
      pltpu.async_copy(y1_hbm.at[sl, :], cb, sem_g1),
      pltpu.async_copy(dinv_hbm.at[sl], dvb, sem_s0),
  ]
  pltpu.sync_copy(b1_hbm, b1b)

  _zero_fill(msg0, rows_per_sub, 1)
  pltpu.sync_copy(msg0.at[pl.ds(0, rows_per_sub), :], acc_sh.at[sl, :])
  for d in stag:
    d.wait()

  b1v = b1b[...]

  def zbody(g, _):
    dv = dvb[pl.ds(g * LANES, LANES)]
    for u in range(LANES):
      i = g * LANES + u
      acc = ab[i, :] + bb[i, :] + cb[i, :]
      ab[i, :] = jnp.maximum(acc * dv[u] + b1v, 0.0) * dv[u]
    return 0

  lax.fori_loop(0, rows_per_sub // LANES, zbody, 0)
  pltpu.sync_copy(ab, y_sh.at[sl, :])

  @pl.when(cid == 0)
  def _():
    pltpu.sync_copy(ab, z_out.at[sl, :])

  plsc.subcore_barrier()

  def issue_gather(t):
    slot = t % 2
    return pltpu.async_copy(y_sh.at[row_all.at[pl.ds(t * CH, CH)]],
                            msgb[slot], sem_g[slot])

  def scale(t):
    slot = t % 2
    buf = msgb[slot]

    def body(g, _):
      wv = w_all[pl.ds(t * CH + g * LANES, LANES)]
      for u in range(LANES):
        i = g * LANES + u
        buf[i, :] = buf[i, :] * wv[u]
      return 0

    lax.fori_loop(0, CH // LANES, body, 0)

  def issue_scatter(t):
    slot = t % 2
    ds = []
    for jj in range(NJ):
      ds.append(pltpu.async_copy(
          msgb[slot].at[pl.ds(jj * 128, 128), :],
          acc_sh.at[col_all.at[t * NJ + jj]], sem_s[slot], add=True))
    return ds

  gath = {0: issue_gather(0)}
  scat = {}
  for t in range(n_chunks):
    if t + 1 < n_chunks:
      if t - 1 >= 0:
        for d in scat.pop(t - 1):
          d.wait()
      gath[t + 1] = issue_gather(t + 1)
    gath.pop(t).wait()
    scale(t)
    scat[t] = issue_scatter(t)
  for ds in scat.values():
    for d in ds:
      d.wait()

  plsc.subcore_barrier()
  pltpu.sync_copy(acc_sh.at[sl, :], p2_out.at[cid, sl, :])


def _sc_agg2(row, col2, w, p1, y1, dinv1d, b1, n_pad, feat):
  e_pad = row.shape[0]
  n_chunks = e_pad // (NW * CH)
  epw = n_chunks * CH
  rps = n_pad // NS
  mesh = plsc.VectorSubcoreMesh(core_axis_name="c", subcore_axis_name="s")
  return pl.kernel(
      functools.partial(_agg2_body, n_pad, n_chunks),
      out_type=(jax.ShapeDtypeStruct((NC, n_pad, feat), jnp.float32),
                jax.ShapeDtypeStruct((n_pad, feat), jnp.float32)),
      mesh=mesh,
      scratch_types=[
          pltpu.VMEM((epw,), jnp.int32),
          pltpu.VMEM((epw // 128, 128), jnp.int32),
          pltpu.VMEM((epw,), jnp.float32),
          pltpu.VMEM((CH, feat), jnp.float32),
          pltpu.VMEM((CH, feat), jnp.float32),
          pltpu.VMEM((rps, feat), jnp.float32),
          pltpu.VMEM((rps, feat), jnp.float32),
          pltpu.VMEM((rps, feat), jnp.float32),
          pltpu.VMEM((rps,), jnp.float32),
          pltpu.VMEM((feat,), jnp.float32),
          pltpu.VMEM_SHARED((n_pad, feat), jnp.float32),
          pltpu.VMEM_SHARED((n_pad, feat), jnp.float32),
          pltpu.SemaphoreType.DMA,
          pltpu.SemaphoreType.DMA,
          pltpu.SemaphoreType.DMA,
          pltpu.SemaphoreType.DMA,
      ],
      compiler_params=pltpu.CompilerParams(use_tc_tiling_on_sc=False),
  )(row, col2, w, p1, y1, dinv1d, b1)


def _sc_deg(col2, w, n_pad):
  e_pad = col2.shape[0] * 128
  mesh = plsc.VectorSubcoreMesh(core_axis_name="c", subcore_axis_name="s")
  return pl.kernel(
      functools.partial(_deg_body, n_pad, e_pad // NW // 128),
      out_type=jax.ShapeDtypeStruct((NC, n_pad), jnp.float32),
      mesh=mesh,
      scratch_types=[
          pltpu.VMEM((e_pad // NW // 128, 128), jnp.int32),
          pltpu.VMEM((e_pad // NW,), jnp.float32),
          pltpu.VMEM((n_pad // NS,), jnp.float32),
          pltpu.VMEM_SHARED((n_pad,), jnp.float32),
          pltpu.SemaphoreType.DMA,
          pltpu.SemaphoreType.DMA,
      ],
  )(col2, w)


def _sc_agg(row, col2, w, y, n_pad, feat):
  e_pad = row.shape[0]
  n_chunks = e_pad // (NW * CH)
  epw = n_chunks * CH
  mesh = plsc.VectorSubcoreMesh(core_axis_name="c", subcore_axis_name="s")
  return pl.kernel(
      functools.partial(_agg_body, n_pad, n_chunks),
      out_type=jax.ShapeDtypeStruct((NC, n_pad, feat), jnp.float32),
      mesh=mesh,
      scratch_types=[
          pltpu.VMEM((epw,), jnp.int32),
          pltpu.VMEM((epw // 128, 128), jnp.int32),
          pltpu.VMEM((epw,), jnp.float32),
          pltpu.VMEM((CH, feat), jnp.float32),
          pltpu.VMEM((CH, feat), jnp.float32),
          pltpu.VMEM_SHARED((n_pad, feat), jnp.float32),
          pltpu.VMEM_SHARED((n_pad, feat), jnp.float32),
          pltpu.SemaphoreType.DMA,
          pltpu.SemaphoreType.DMA,
          pltpu.SemaphoreType.DMA,
          pltpu.SemaphoreType.DMA,
      ],
      compiler_params=pltpu.CompilerParams(use_tc_tiling_on_sc=False),
  )(row, col2, w, y)


def _tc_a_body(n, deg_ref, x_ref, w1_ref, dinv_ref, y1_ref):
  deg = deg_ref[0] + deg_ref[1] + 1.0
  dinv = lax.rsqrt(deg)
  dinv_ref[...] = dinv
  xw = jnp.dot(x_ref[...], w1_ref[...], preferred_element_type=jnp.float32)
  y1_ref[pl.ds(0, n), :] = xw * dinv[:n]


def _tc_b_body(p1_ref, y1_ref, dinv_ref, b1_ref, z_ref):
  dinv = dinv_ref[...]
  agg = p1_ref[0] + p1_ref[1] + y1_ref[...]
  h = jnp.maximum(agg * dinv + b1_ref[...], 0.0)
  z_ref[...] = h * dinv


def _tc_c_body(p2_ref, z_ref, dinv_ref, b2_ref, w2_ref, out_ref):
  agg = p2_ref[0] + p2_ref[1] + z_ref[...]
  aw = jnp.dot(agg, w2_ref[...], preferred_element_type=jnp.float32)
  out_ref[...] = aw * dinv_ref[...] + b2_ref[...]


def kernel(x, edge_index, edge_weight, W1, b1, W2, b2):
  n, _ = x.shape
  e = edge_index.shape[1]
  hid = W1.shape[1]
  ncls = W2.shape[1]

  n_pad = ((n + NW * LANES - 1) // (NW * LANES)) * (NW * LANES)
  e_pad = ((e + NW * CH - 1) // (NW * CH)) * (NW * CH)

  row = jnp.concatenate([edge_index[0],
                         jnp.zeros((e_pad - e,), jnp.int32)])
  colp = jnp.concatenate([edge_index[1],
                          jnp.zeros((e_pad - e,), jnp.int32)])
  col2 = colp.reshape(e_pad // 128, 128)
  w = jnp.concatenate([edge_weight, jnp.zeros((e_pad - e,), jnp.float32)])
  b1r = b1.reshape(1, hid)
  b2r = b2.reshape(1, ncls)

  deg = _sc_deg(col2, w, n_pad)
  deg3 = deg.reshape(NC, n_pad, 1)

  dinv, y1 = pl.pallas_call(
      functools.partial(_tc_a_body, n),
      out_shape=(jax.ShapeDtypeStruct((n_pad, 1), jnp.float32),
                 jax.ShapeDtypeStruct((n_pad, hid), jnp.float32)),
  )(deg3, x, W1)

  p1 = _sc_agg(row, col2, w, y1, n_pad, hid)

  p2, z = _sc_agg2(row, col2, w, p1, y1, dinv.reshape(n_pad), b1,
                   n_pad, hid)

  out = pl.pallas_call(
      _tc_c_body,
      out_shape=jax.ShapeDtypeStruct((n_pad, ncls), jnp.float32),
  )(p2, z, dinv, b2r, W2)

  return out[:n, :ncls]


# speedup vs baseline: 68.3754x; 1.1374x over previous
import functools

import jax
import jax.numpy as jnp
from jax import lax
from jax.experimental import pallas as pl
from jax.experimental.pallas import tpu as pltpu
from jax.experimental.pallas import tpu_sc as plsc

NC = 2
NS = 16
NW = NC * NS
LANES = 16
CH = 1024
NJ = CH // 128


def _flat_worker_id():
  return lax.axis_index("s") * NC + lax.axis_index("c")


def _zero_fill(ref, n_rows, n_vregs):
  z = jnp.zeros((LANES,), jnp.float32)

  def body(i, _):
    if n_vregs == 0:
      ref[pl.ds(i * LANES, LANES)] = z
    else:
      for k in range(n_vregs):
        ref[i, pl.ds(k * LANES, LANES)] = z
    return 0

  count = n_rows // LANES if n_vregs == 0 else n_rows
  lax.fori_loop(0, count, body, 0, unroll=8)


def _row_scale(buf, dvb, n_rows):
  def body(g, _):
    dv = dvb[pl.ds(g * LANES, LANES)]
    for u in range(LANES):
      i = g * LANES + u
      buf[i, :] = buf[i, :] * dv[u]
    return 0

  lax.fori_loop(0, n_rows // LANES, body, 0)


def _deg_body(n_pad, n_jrows, ei3_hbm, w_hbm, out_hbm, col_all, w_all, zb,
              deg_sh, sem_a, sem_b):
  cid = lax.axis_index("c")
  sid = lax.axis_index("s")
  wid = _flat_worker_id()
  rows_per_sub = n_pad // NS
  epw = n_jrows * 128

  base = wid * epw
  crow = pl.multiple_of(base // 128, 8)
  d_col = pltpu.async_copy(ei3_hbm.at[1, pl.ds(crow, n_jrows), :], col_all,
                           sem_a)
  d_w = pltpu.async_copy(w_hbm.at[pl.ds(base, epw)], w_all, sem_b)

  _zero_fill(zb, rows_per_sub, 0)
  pltpu.sync_copy(zb, deg_sh.at[pl.ds(sid * rows_per_sub, rows_per_sub)])
  d_col.wait()
  d_w.wait()
  plsc.subcore_barrier()

  pend = []
  for j in range(n_jrows):
    if len(pend) >= 16:
      pend.pop(0).wait()
    pend.append(pltpu.async_copy(w_all.at[pl.ds(j * 128, 128)],
                                 deg_sh.at[col_all.at[j]], sem_a, add=True))
  for d in pend:
    d.wait()

  plsc.subcore_barrier()
  sl = pl.ds(sid * rows_per_sub, rows_per_sub)
  pltpu.sync_copy(deg_sh.at[sl], out_hbm.at[cid, sl])


def _edge_pipeline(n_chunks, row_all, col_all, w_all, msgb, y_sh, acc_sh,
                   sem_g, sem_s):
  def issue_gather(t):
    slot = t % 2
    ds = []
    for jj in range(NJ):
      ds.append(pltpu.async_copy(
          y_sh.at[row_all.at[t * NJ + jj]],
          msgb[slot].at[pl.ds(jj * 128, 128), :], sem_g[slot]))
    return ds

  def scale(t):
    slot = t % 2
    buf = msgb[slot]

    def body(g, _):
      wv = w_all[pl.ds(t * CH + g * LANES, LANES)]
      for u in range(LANES):
        i = g * LANES + u
        buf[i, :] = buf[i, :] * wv[u]
      return 0

    lax.fori_loop(0, CH // LANES, body, 0)

  def issue_scatter(t):
    slot = t % 2
    ds = []
    for jj in range(NJ):
      ds.append(pltpu.async_copy(
          msgb[slot].at[pl.ds(jj * 128, 128), :],
          acc_sh.at[col_all.at[t * NJ + jj]], sem_s[slot], add=True))
    return ds

  gath = {0: issue_gather(0)}
  scat = {}
  for t in range(n_chunks):
    if t + 1 < n_chunks:
      if t - 1 >= 0:
        for d in scat.pop(t - 1):
          d.wait()
      gath[t + 1] = issue_gather(t + 1)
    for d in gath.pop(t):
      d.wait()
    scale(t)
    scat[t] = issue_scatter(t)
  for ds in scat.values():
    for d in ds:
      d.wait()


def _agg1_body(n_pad, n_chunks, ei3_hbm, w_hbm, xw_hbm, dinv_hbm,
               p1_out, y1_out,
               row_all, col_all, w_all, msg0, msg1, ab, dvb,
               y_sh, acc_sh, sem_g0, sem_g1, sem_s0, sem_s1):
  cid = lax.axis_index("c")
  sid = lax.axis_index("s")
  wid = _flat_worker_id()
  rows_per_sub = n_pad // NS
  epw = n_chunks * CH

  base = wid * epw
  crow = pl.multiple_of(base // 128, 8)
  sl = pl.ds(sid * rows_per_sub, rows_per_sub)
  stag = [
      pltpu.async_copy(ei3_hbm.at[0, pl.ds(crow, epw // 128), :], row_all,
                       sem_g0),
      pltpu.async_copy(ei3_hbm.at[1, pl.ds(crow, epw // 128), :], col_all,
                       sem_g1),
      pltpu.async_copy(w_hbm.at[pl.ds(base, epw)], w_all, sem_s0),
      pltpu.async_copy(xw_hbm.at[sl, :], ab, sem_s1),
      pltpu.async_copy(dinv_hbm.at[sl], dvb, sem_g0),
  ]

  _zero_fill(msg0, rows_per_sub, 1)
  pltpu.sync_copy(msg0.at[pl.ds(0, rows_per_sub), :], acc_sh.at[sl, :])
  for d in stag:
    d.wait()

  _row_scale(ab, dvb, rows_per_sub)
  pltpu.sync_copy(ab, y_sh.at[sl, :])

  @pl.when(cid == 0)
  def _():
    pltpu.sync_copy(ab, y1_out.at[sl, :])

  plsc.subcore_barrier()
  _edge_pipeline(n_chunks, row_all, col_all, w_all, (msg0, msg1), y_sh,
                 acc_sh, (sem_g0, sem_g1), (sem_s0, sem_s1))
  plsc.subcore_barrier()
  pltpu.sync_copy(acc_sh.at[sl, :], p1_out.at[cid, sl, :])


def _agg2_body(n_pad, n_chunks, ei3_hbm, w_hbm, p1_hbm, y1_hbm,
               dinv_hbm, b1_hbm, q_out, zq_out,
               row_all, col_all, w_all, msg0, msg1, ab, bb, cb, dvb, b1b,
               y_sh, acc_sh, sem_g0, sem_g1, sem_s0, sem_s1):
  cid = lax.axis_index("c")
  sid = lax.axis_index("s")
  wid = _flat_worker_id()
  rows_per_sub = n_pad // NS
  epw = n_chunks * CH

  base = wid * epw
  crow = pl.multiple_of(base // 128, 8)
  sl = pl.ds(sid * rows_per_sub, rows_per_sub)
  stag = [
      pltpu.async_copy(ei3_hbm.at[0, pl.ds(crow, epw // 128), :], row_all,
                       sem_g0),
      pltpu.async_copy(ei3_hbm.at[1, pl.ds(crow, epw // 128), :], col_all,
                       sem_g1),
      pltpu.async_copy(w_hbm.at[pl.ds(base, epw)], w_all, sem_s0),
      pltpu.async_copy(p1_hbm.at[0, sl, :], ab, sem_s1),
      pltpu.async_copy(p1_hbm.at[1, sl, :], bb, sem_g0),
      pltpu.async_copy(y1_hbm.at[sl, :], cb, sem_g1),
      pltpu.async_copy(dinv_hbm.at[sl], dvb, sem_s0),
  ]
  pltpu.sync_copy(b1_hbm, b1b)

  _zero_fill(msg0, rows_per_sub, 1)
  pltpu.sync_copy(msg0.at[pl.ds(0, rows_per_sub), :], acc_sh.at[sl, :])
  for d in stag:
    d.wait()

  b1v = b1b[...]

  def zbody(g, _):
    dv = dvb[pl.ds(g * LANES, LANES)]
    for u in range(LANES):
      i = g * LANES + u
      acc = ab[i, :] + bb[i, :] + cb[i, :]
      ab[i, :] = jnp.maximum(acc * dv[u] + b1v, 0.0) * dv[u]
    return 0

  lax.fori_loop(0, rows_per_sub // LANES, zbody, 0)
  pltpu.sync_copy(ab, y_sh.at[sl, :])

  _row_scale(ab, dvb, rows_per_sub)

  @pl.when(cid == 0)
  def _():
    pltpu.sync_copy(ab, zq_out.at[sl, :])

  plsc.subcore_barrier()
  _edge_pipeline(n_chunks, row_all, col_all, w_all, (msg0, msg1), y_sh,
                 acc_sh, (sem_g0, sem_g1), (sem_s0, sem_s1))
  plsc.subcore_barrier()

  pltpu.sync_copy(acc_sh.at[sl, :], bb)
  _row_scale(bb, dvb, rows_per_sub)
  pltpu.sync_copy(bb, q_out.at[cid, sl, :])


def _sc_deg(ei3, w, n_pad):
  e_pad = w.shape[0]
  mesh = plsc.VectorSubcoreMesh(core_axis_name="c", subcore_axis_name="s")
  return pl.kernel(
      functools.partial(_deg_body, n_pad, e_pad // NW // 128),
      out_type=jax.ShapeDtypeStruct((NC, n_pad), jnp.float32),
      mesh=mesh,
      scratch_types=[
          pltpu.VMEM((e_pad // NW // 128, 128), jnp.int32),
          pltpu.VMEM((e_pad // NW,), jnp.float32),
          pltpu.VMEM((n_pad // NS,), jnp.float32),
          pltpu.VMEM_SHARED((n_pad,), jnp.float32),
          pltpu.SemaphoreType.DMA,
          pltpu.SemaphoreType.DMA,
      ],
  )(ei3, w)


def _sc_agg1(ei3, w, xw, dinv1d, n_pad, feat):
  e_pad = w.shape[0]
  n_chunks = e_pad // (NW * CH)
  epw = n_chunks * CH
  rps = n_pad // NS
  mesh = plsc.VectorSubcoreMesh(core_axis_name="c", subcore_axis_name="s")
  return pl.kernel(
      functools.partial(_agg1_body, n_pad, n_chunks),
      out_type=(jax.ShapeDtypeStruct((NC, n_pad, feat), jnp.float32),
                jax.ShapeDtypeStruct((n_pad, feat), jnp.float32)),
      mesh=mesh,
      scratch_types=[
          pltpu.VMEM((epw // 128, 128), jnp.int32),
          pltpu.VMEM((epw // 128, 128), jnp.int32),
          pltpu.VMEM((epw,), jnp.float32),
          pltpu.VMEM((CH, feat), jnp.float32),
          pltpu.VMEM((CH, feat), jnp.float32),
          pltpu.VMEM((rps, feat), jnp.float32),
          pltpu.VMEM((rps,), jnp.float32),
          pltpu.VMEM_SHARED((n_pad, feat), jnp.float32),
          pltpu.VMEM_SHARED((n_pad, feat), jnp.float32),
          pltpu.SemaphoreType.DMA,
          pltpu.SemaphoreType.DMA,
          pltpu.SemaphoreType.DMA,
          pltpu.SemaphoreType.DMA,
      ],
      compiler_params=pltpu.CompilerParams(use_tc_tiling_on_sc=False),
  )(ei3, w, xw, dinv1d)


def _sc_agg2(ei3, w, p1, y1, dinv1d, b1, n_pad, feat):
  e_pad = w.shape[0]
  n_chunks = e_pad // (NW * CH)
  epw = n_chunks * CH
  rps = n_pad // NS
  mesh = plsc.VectorSubcoreMesh(core_axis_name="c", subcore_axis_name="s")
  return pl.kernel(
      functools.partial(_agg2_body, n_pad, n_chunks),
      out_type=(jax.ShapeDtypeStruct((NC, n_pad, feat), jnp.float32),
                jax.ShapeDtypeStruct((n_pad, feat), jnp.float32)),
      mesh=mesh,
      scratch_types=[
          pltpu.VMEM((epw // 128, 128), jnp.int32),
          pltpu.VMEM((epw // 128, 128), jnp.int32),
          pltpu.VMEM((epw,), jnp.float32),
          pltpu.VMEM((CH, feat), jnp.float32),
          pltpu.VMEM((CH, feat), jnp.float32),
          pltpu.VMEM((rps, feat), jnp.float32),
          pltpu.VMEM((rps, feat), jnp.float32),
          pltpu.VMEM((rps, feat), jnp.float32),
          pltpu.VMEM((rps,), jnp.float32),
          pltpu.VMEM((feat,), jnp.float32),
          pltpu.VMEM_SHARED((n_pad, feat), jnp.float32),
          pltpu.VMEM_SHARED((n_pad, feat), jnp.float32),
          pltpu.SemaphoreType.DMA,
          pltpu.SemaphoreType.DMA,
          pltpu.SemaphoreType.DMA,
          pltpu.SemaphoreType.DMA,
      ],
      compiler_params=pltpu.CompilerParams(use_tc_tiling_on_sc=False),
  )(ei3, w, p1, y1, dinv1d, b1)


def _tc_a_body(n, deg_ref, x_ref, w1_ref, dinv_ref, xw_ref):
  d = deg_ref[0] + deg_ref[1] + 1.0
  dinv_ref[...] = lax.rsqrt(d)
  xw_ref[pl.ds(0, n), :] = jnp.dot(x_ref[...], w1_ref[...],
                                   preferred_element_type=jnp.float32)


def _tc_c_body(n, q_ref, zq_ref, b2_ref, w2_ref, out_ref):
  a = (q_ref[0] + q_ref[1] + zq_ref[...])[:n]
  out_ref[...] = jnp.dot(a, w2_ref[...],
                         preferred_element_type=jnp.float32) + b2_ref[...]


def kernel(x, edge_index, edge_weight, W1, b1, W2, b2):
  n, _ = x.shape
  e = edge_index.shape[1]
  hid = W1.shape[1]
  ncls = W2.shape[1]

  n_pad = ((n + NW * LANES - 1) // (NW * LANES)) * (NW * LANES)
  e_pad = ((e + NW * CH - 1) // (NW * CH)) * (NW * CH)

  ei = jnp.pad(edge_index, ((0, 0), (0, e_pad - e)))
  ei3 = ei.reshape(2, e_pad // 128, 128)
  w = jnp.concatenate([edge_weight, jnp.zeros((e_pad - e,), jnp.float32)])
  b2r = b2.reshape(1, ncls)

  deg = _sc_deg(ei3, w, n_pad)

  dinv1d, xw = pl.pallas_call(
      functools.partial(_tc_a_body, n),
      out_shape=(jax.ShapeDtypeStruct((n_pad,), jnp.float32),
                 jax.ShapeDtypeStruct((n_pad, hid), jnp.float32)),
  )(deg, x, W1)

  p1, y1 = _sc_agg1(ei3, w, xw, dinv1d, n_pad, hid)
  q, zq = _sc_agg2(ei3, w, p1, y1, dinv1d, b1, n_pad, hid)

  out = pl.pallas_call(
      functools.partial(_tc_c_body, n),
      out_shape=jax.ShapeDtypeStruct((n, ncls), jnp.float32),
  )(q, zq, b2r, W2)

  return out
